# baseline (speedup 1.0000x reference)
import jax
import jax.numpy as jnp
from jax import lax
from jax.experimental import pallas as pl
from jax.experimental.pallas import tpu as pltpu

B, T, DM, H, D, BT = 4, 2048, 1024, 16, 64, 16
ETA_BASE = 0.01
LN_EPS = 1e-05
THETA = 10000.0
NC = T // BT
G = (B * H) // 2


def _ttt_body(k_ref, v_ref, q_ref, e_ref, w0_ref, lnw_ref, lnb_ref,
              z_ref, wf_ref, tk_s, ue_s, tq_s):
    c = pl.program_id(1)

    @pl.when(c == 0)
    def _init():
        wf_ref[...] = w0_ref[...]

    row = lax.broadcasted_iota(jnp.int32, (BT, BT), 0)
    col = lax.broadcasted_iota(jnp.int32, (BT, BT), 1)
    mask = row > col

    def _ln_stats(a):
        mu = jnp.mean(a, axis=-1, keepdims=True)
        ac = a - mu
        var = jnp.mean(ac * ac, axis=-1, keepdims=True)
        inv = lax.rsqrt(var + LN_EPS)
        return ac * inv, inv

    for hh in range(G):
        tk_s[pl.ds(hh * BT, BT), :] = lax.dot_general(
            k_ref[hh], wf_ref[hh], (((1,), (1,)), ((), ())),
            preferred_element_type=jnp.float32)

    lnw = lnw_ref[...]
    lnb = lnb_ref[...]
    Kall = k_ref[...].reshape(G * BT, D)
    Vall = v_ref[...].reshape(G * BT, D)
    tK = tk_s[...]
    xhat, inv = _ln_stats(tK)
    g = (2.0 / BT) * (Kall + (xhat * lnw + lnb) - Vall)
    dxhat = g * lnw
    u = (dxhat
         - jnp.mean(dxhat, axis=-1, keepdims=True)
         - xhat * jnp.mean(dxhat * xhat, axis=-1, keepdims=True)) * inv
    ue_s[...] = u * e_ref[0, 0]

    for hh in range(G):
        Qb = q_ref[hh]
        S = lax.dot_general(Qb, k_ref[hh], (((1,), (1,)), ((), ())),
                            preferred_element_type=jnp.float32)
        corr = jnp.dot(jnp.where(mask, S, 0.0), ue_s[pl.ds(hh * BT, BT), :],
                       preferred_element_type=jnp.float32)
        tq_s[pl.ds(hh * BT, BT), :] = lax.dot_general(
            Qb, wf_ref[hh], (((1,), (1,)), ((), ())),
            preferred_element_type=jnp.float32) - corr

    qhat, _ = _ln_stats(tq_s[...])
    Qall = q_ref[...].reshape(G * BT, D)
    z_ref[...] = (Qall + (qhat * lnw + lnb)).reshape(G, BT, D)

    for hh in range(G):
        wf_ref[hh] = wf_ref[hh] - lax.dot_general(
            ue_s[pl.ds(hh * BT, BT), :], k_ref[hh], (((0,), (0,)), ((), ())),
            preferred_element_type=jnp.float32)


@jax.jit
def kernel(x, W, Wq, Wk, Wv, Wo, Wlr, ln_w, ln_b):
    xf = x.reshape(B * T, DM)
    Wcat = jnp.concatenate([Wq, Wk, Wv], axis=0)
    qkve = xf @ Wcat.T
    q = qkve[:, 0 * DM:1 * DM].reshape(B, T, H, D).transpose(0, 2, 1, 3)
    k = qkve[:, 1 * DM:2 * DM].reshape(B, T, H, D).transpose(0, 2, 1, 3)
    v = qkve[:, 2 * DM:3 * DM].reshape(B, T, H, D).transpose(0, 2, 1, 3)

    pos = jnp.arange(T, dtype=jnp.float32)
    inv_freq = 1.0 / (THETA ** (jnp.arange(0, D, 2, dtype=jnp.float32) / D))
    freqs = pos[:, None] * inv_freq[None, :]
    emb = jnp.concatenate([freqs, freqs], axis=-1)
    cos = jnp.cos(emb)[None, None]
    sin = jnp.sin(emb)[None, None]

    def rot(a):
        a1, a2 = jnp.split(a, 2, axis=-1)
        return jnp.concatenate([-a2, a1], axis=-1)

    q = (q * cos + rot(q) * sin).reshape(B * H, T, D)
    k = (k * cos + rot(k) * sin).reshape(B * H, T, D)
    v = v.reshape(B * H, T, D)

    et = (ETA_BASE * jax.nn.sigmoid(xf @ Wlr.T)).reshape(2, 2, NC, 1, BT)
    e4 = jnp.broadcast_to(et, (2, 2, NC, H, BT)).transpose(0, 2, 1, 3, 4)
    e4 = jnp.broadcast_to(e4.reshape(2, NC, G * BT, 1), (2, NC, G * BT, D))

    w0 = W.reshape(B * H, D, D)
    lnw_big = jnp.tile(jnp.repeat(ln_w, BT, axis=0), (G // H, 1))
    lnb_big = jnp.tile(jnp.repeat(ln_b, BT, axis=0), (G // H, 1))

    z, wf = pl.pallas_call(
        _ttt_body,
        grid=(2, NC),
        in_specs=[
            pl.BlockSpec((G, BT, D), lambda i, c: (i, c, 0)),
            pl.BlockSpec((G, BT, D), lambda i, c: (i, c, 0)),
            pl.BlockSpec((G, BT, D), lambda i, c: (i, c, 0)),
            pl.BlockSpec((1, 1, G * BT, D), lambda i, c: (i, c, 0, 0)),
            pl.BlockSpec((G, D, D), lambda i, c: (i, 0, 0)),
            pl.BlockSpec((G * BT, D), lambda i, c: (0, 0)),
            pl.BlockSpec((G * BT, D), lambda i, c: (0, 0)),
        ],
        out_specs=[
            pl.BlockSpec((G, BT, D), lambda i, c: (i, c, 0)),
            pl.BlockSpec((G, D, D), lambda i, c: (i, 0, 0)),
        ],
        out_shape=[
            jax.ShapeDtypeStruct((B * H, T, D), jnp.float32),
            jax.ShapeDtypeStruct((B * H, D, D), jnp.float32),
        ],
        scratch_shapes=[
            pltpu.VMEM((G * BT, D), jnp.float32),
            pltpu.VMEM((G * BT, D), jnp.float32),
            pltpu.VMEM((G * BT, D), jnp.float32),
        ],
        compiler_params=pltpu.CompilerParams(
            dimension_semantics=("parallel", "arbitrary")),
    )(k, v, q, e4, w0, lnw_big, lnb_big)

    zt = z.reshape(B, H, T, D).transpose(0, 2, 1, 3).reshape(B * T, DM)
    out = (zt @ Wo.T).reshape(B, T, DM)
    return out, wf.reshape(B, H, D, D)


# speedup vs baseline: 1.2388x; 1.2388x over previous
import jax
import jax.numpy as jnp
from jax import lax
from jax.experimental import pallas as pl
from jax.experimental.pallas import tpu as pltpu

B, T, DM, H, D, BT = 4, 2048, 1024, 16, 64, 16
ETA_BASE = 0.01
LN_EPS = 1e-05
THETA = 10000.0
NC = T // BT
G = (B * H) // 2


def _ttt_body(k_ref, v_ref, q_ref, e_ref, w0_ref, lnw_ref, lnb_ref,
              z_ref, wf_ref, tk_s, ue_s, tq_s):
    c = pl.program_id(1)

    @pl.when(c == 0)
    def _init():
        wf_ref[...] = w0_ref[...]

    row = lax.broadcasted_iota(jnp.int32, (BT, BT), 0)
    col = lax.broadcasted_iota(jnp.int32, (BT, BT), 1)
    mask = row > col

    def _ln_stats(a):
        mu = jnp.mean(a, axis=-1, keepdims=True)
        ac = a - mu
        var = jnp.mean(ac * ac, axis=-1, keepdims=True)
        inv = lax.rsqrt(var + LN_EPS)
        return ac * inv, inv

    for b in range(2):
        for h in range(H):
            tk_s[b, :, h, :] = lax.dot_general(
                k_ref[b, :, h, :], wf_ref[b * H + h], (((1,), (1,)), ((), ())),
                preferred_element_type=jnp.float32)

    lnw = lnw_ref[...]
    lnb = lnb_ref[...]
    Kall = k_ref[...].reshape(G * BT, D)
    Vall = v_ref[...].reshape(G * BT, D)
    tK = tk_s[...].reshape(G * BT, D)
    xhat, inv = _ln_stats(tK)
    g = (2.0 / BT) * (Kall + (xhat * lnw + lnb) - Vall)
    dxhat = g * lnw
    u = (dxhat
         - jnp.mean(dxhat, axis=-1, keepdims=True)
         - xhat * jnp.mean(dxhat * xhat, axis=-1, keepdims=True)) * inv
    ue_s[...] = (u * e_ref[0, 0]).reshape(2, BT, H, D)

    for b in range(2):
        for h in range(H):
            Qb = q_ref[b, :, h, :]
            S = lax.dot_general(Qb, k_ref[b, :, h, :], (((1,), (1,)), ((), ())),
                                preferred_element_type=jnp.float32)
            corr = jnp.dot(jnp.where(mask, S, 0.0), ue_s[b, :, h, :],
                           preferred_element_type=jnp.float32)
            tq_s[b, :, h, :] = lax.dot_general(
                Qb, wf_ref[b * H + h], (((1,), (1,)), ((), ())),
                preferred_element_type=jnp.float32) - corr

    qhat, _ = _ln_stats(tq_s[...].reshape(G * BT, D))
    Qall = q_ref[...].reshape(G * BT, D)
    z_ref[...] = (Qall + (qhat * lnw + lnb)).reshape(2, BT, H, D)

    for b in range(2):
        for h in range(H):
            wf_ref[b * H + h] = wf_ref[b * H + h] - lax.dot_general(
                ue_s[b, :, h, :], k_ref[b, :, h, :], (((0,), (0,)), ((), ())),
                preferred_element_type=jnp.float32)


@jax.jit
def kernel(x, W, Wq, Wk, Wv, Wo, Wlr, ln_w, ln_b):
    xf = x.reshape(B * T, DM)
    q = (xf @ Wq.T).reshape(B, T, H, D)
    k = (xf @ Wk.T).reshape(B, T, H, D)
    v = (xf @ Wv.T).reshape(B, T, H, D)

    pos = jnp.arange(T, dtype=jnp.float32)
    inv_freq = 1.0 / (THETA ** (jnp.arange(0, D, 2, dtype=jnp.float32) / D))
    freqs = pos[:, None] * inv_freq[None, :]
    emb = jnp.concatenate([freqs, freqs], axis=-1)
    cos = jnp.cos(emb)[None, :, None, :]
    sin = jnp.sin(emb)[None, :, None, :]

    def rot(a):
        a1, a2 = jnp.split(a, 2, axis=-1)
        return jnp.concatenate([-a2, a1], axis=-1)

    q = q * cos + rot(q) * sin
    k = k * cos + rot(k) * sin

    et = (ETA_BASE * jax.nn.sigmoid(xf @ Wlr.T)).reshape(2, 2, NC, BT, 1)
    e4 = jnp.broadcast_to(et, (2, 2, NC, BT, H)).transpose(0, 2, 1, 3, 4)
    e4 = jnp.broadcast_to(e4.reshape(2, NC, G * BT, 1), (2, NC, G * BT, D))

    w0 = W.reshape(B * H, D, D)
    lnw_big = jnp.tile(ln_w, (G * BT // H, 1))
    lnb_big = jnp.tile(ln_b, (G * BT // H, 1))

    z, wf = pl.pallas_call(
        _ttt_body,
        grid=(2, NC),
        in_specs=[
            pl.BlockSpec((2, BT, H, D), lambda i, c: (i, c, 0, 0)),
            pl.BlockSpec((2, BT, H, D), lambda i, c: (i, c, 0, 0)),
            pl.BlockSpec((2, BT, H, D), lambda i, c: (i, c, 0, 0)),
            pl.BlockSpec((1, 1, G * BT, D), lambda i, c: (i, c, 0, 0)),
            pl.BlockSpec((G, D, D), lambda i, c: (i, 0, 0)),
            pl.BlockSpec((G * BT, D), lambda i, c: (0, 0)),
            pl.BlockSpec((G * BT, D), lambda i, c: (0, 0)),
        ],
        out_specs=[
            pl.BlockSpec((2, BT, H, D), lambda i, c: (i, c, 0, 0)),
            pl.BlockSpec((G, D, D), lambda i, c: (i, 0, 0)),
        ],
        out_shape=[
            jax.ShapeDtypeStruct((B, T, H, D), jnp.float32),
            jax.ShapeDtypeStruct((B * H, D, D), jnp.float32),
        ],
        scratch_shapes=[
            pltpu.VMEM((2, BT, H, D), jnp.float32),
            pltpu.VMEM((2, BT, H, D), jnp.float32),
            pltpu.VMEM((2, BT, H, D), jnp.float32),
        ],
        compiler_params=pltpu.CompilerParams(
            dimension_semantics=("parallel", "arbitrary")),
    )(k, v, q, e4, w0, lnw_big, lnb_big)

    out = (z.reshape(B * T, DM) @ Wo.T).reshape(B, T, DM)
    return out, wf.reshape(B, H, D, D)
